# Initial kernel scaffold; baseline (speedup 1.0000x reference)
import jax
import jax.numpy as jnp
from jax.experimental import pallas as pl


def kernel(x, edge_index, W1, b1, W2, b2, fc1_W, fc1_b, fc2_W, fc2_b, fc3_W, fc3_b, fc4_W, fc4_b):
    raise NotImplementedError("write your pallas kernel here")


# speedup vs baseline: 9.4301x; 9.4301x over previous
import functools

import jax
import jax.numpy as jnp
from jax import lax
from jax.experimental import pallas as pl
from jax.experimental.pallas import tpu as pltpu
from jax.experimental.pallas import tpu_sc as plsc

NC, NS = 2, 16
NW = NC * NS
CH = 128

_mesh = lambda: plsc.VectorSubcoreMesh(core_axis_name="c", subcore_axis_name="s")


def _row_partition(n):
    rpt = (n // NS) & ~7
    tail = n - rpt * NS
    assert tail % 8 == 0
    return rpt, tail


def _copy_rows(s, src, dst, rpt, tail, add=False):
    pltpu.sync_copy(src.at[pl.ds(s * rpt, rpt)],
                    dst.at[pl.ds(s * rpt, rpt)], add=add)
    if tail:
        @pl.when(s == NS - 1)
        def _():
            pltpu.sync_copy(src.at[pl.ds(NS * rpt, tail)],
                            dst.at[pl.ds(NS * rpt, tail)], add=add)


def _sc_degree(e_dst, ones, zeros, n, d):
    E = e_dst.shape[0]
    epw = E // NW
    nfull, rem = divmod(epw, CH)
    rpt, tail = _row_partition(n)
    assert E % NW == 0 and rem % 8 == 0 and epw % 8 == 0

    @functools.partial(
        pl.kernel, mesh=_mesh(),
        out_type=jax.ShapeDtypeStruct((NC, n, d), jnp.float32),
        scratch_types=[
            pltpu.VMEM((CH,), jnp.int32),
            pltpu.VMEM((max(rem, 8),), jnp.int32),
            pltpu.VMEM((CH, d), jnp.float32),
            pltpu.VMEM_SHARED((n, d), jnp.float32),
        ],
    )
    def k(edst_hbm, ones_hbm, zeros_hbm, out_hbm, dstv, dstvr, ones_v, acc):
        c = lax.axis_index("c")
        s = lax.axis_index("s")
        wid = s * NC + c
        _copy_rows(s, zeros_hbm, acc, rpt, tail)
        pltpu.sync_copy(ones_hbm, ones_v)
        plsc.subcore_barrier()
        base0 = wid * epw

        def step(j, carry):
            b = base0 + j * CH
            pltpu.sync_copy(edst_hbm.at[pl.ds(b, CH)], dstv)
            pltpu.sync_copy(ones_v, acc.at[dstv], add=True)
            return carry

        lax.fori_loop(0, nfull, step, 0)
        if rem:
            b = base0 + nfull * CH
            pltpu.sync_copy(edst_hbm.at[pl.ds(b, rem)], dstvr)
            pltpu.sync_copy(ones_v.at[pl.ds(0, rem)], acc.at[dstvr], add=True)
        plsc.subcore_barrier()
        _copy_rows(s, acc, out_hbm.at[c], rpt, tail)

    return k(e_dst, ones, zeros)


def _sc_aggregate(table, e_src, e_dst, zeros, n, d):
    E = e_src.shape[0]
    epw = E // NW
    nfull, rem = divmod(epw, CH)
    rpt, tail = _row_partition(n)
    assert E % NW == 0 and rem % 8 == 0 and epw % 8 == 0

    @functools.partial(
        pl.kernel, mesh=_mesh(),
        out_type=jax.ShapeDtypeStruct((NC, n, d), jnp.float32),
        scratch_types=[
            pltpu.VMEM((CH,), jnp.int32),
            pltpu.VMEM((CH,), jnp.int32),
            pltpu.VMEM((max(rem, 8),), jnp.int32),
            pltpu.VMEM((max(rem, 8),), jnp.int32),
            pltpu.VMEM((CH, d), jnp.float32),
            pltpu.SemaphoreType.DMA,
            pltpu.VMEM_SHARED((n, d), jnp.float32),
        ],
    )
    def k(tab_hbm, esrc_hbm, edst_hbm, zeros_hbm, out_hbm,
          srcv, dstv, srcvr, dstvr, rows, sem, acc):
        c = lax.axis_index("c")
        s = lax.axis_index("s")
        wid = s * NC + c
        _copy_rows(s, zeros_hbm, acc, rpt, tail)
        plsc.subcore_barrier()
        base0 = wid * epw

        def step(j, carry):
            b = base0 + j * CH
            pltpu.sync_copy(esrc_hbm.at[pl.ds(b, CH)], srcv)
            pltpu.sync_copy(edst_hbm.at[pl.ds(b, CH)], dstv)
            pltpu.async_copy(tab_hbm.at[srcv], rows, sem).wait()
            pltpu.sync_copy(rows, acc.at[dstv], add=True)
            return carry

        lax.fori_loop(0, nfull, step, 0)
        if rem:
            b = base0 + nfull * CH
            pltpu.sync_copy(esrc_hbm.at[pl.ds(b, rem)], srcvr)
            pltpu.sync_copy(edst_hbm.at[pl.ds(b, rem)], dstvr)
            rows_r = rows.at[pl.ds(0, rem)]
            pltpu.async_copy(tab_hbm.at[srcvr], rows_r, sem).wait()
            pltpu.sync_copy(rows_r, acc.at[dstvr], add=True)
        plsc.subcore_barrier()
        _copy_rows(s, acc, out_hbm.at[c], rpt, tail)

    return k(table, e_src, e_dst, zeros)


def _sc_edge_gather(a0, a1, e_src, e_dst, d):
    E = e_src.shape[0]
    epw = E // NW
    nfull, rem = divmod(epw, CH)
    assert E % NW == 0 and rem % 8 == 0 and epw % 8 == 0 and d % 16 == 0
    dl = d // 16

    @functools.partial(
        pl.kernel, mesh=_mesh(),
        out_type=jax.ShapeDtypeStruct((E, d), jnp.float32),
        scratch_types=[
            pltpu.VMEM((CH,), jnp.int32),
            pltpu.VMEM((CH,), jnp.int32),
            pltpu.VMEM((max(rem, 8),), jnp.int32),
            pltpu.VMEM((max(rem, 8),), jnp.int32),
            pltpu.VMEM((CH, d), jnp.float32),
            pltpu.VMEM((CH, d), jnp.float32),
            pltpu.SemaphoreType.DMA,
            pltpu.SemaphoreType.DMA,
        ],
    )
    def k(a0_hbm, a1_hbm, esrc_hbm, edst_hbm, out_hbm,
          i0, i1, i0r, i1r, ra, rb, sema, semb):
        c = lax.axis_index("c")
        s = lax.axis_index("s")
        wid = s * NC + c
        base0 = wid * epw

        def do_chunk(b, iv0, iv1, sz):
            pltpu.sync_copy(esrc_hbm.at[pl.ds(b, sz)], iv0)
            pltpu.sync_copy(edst_hbm.at[pl.ds(b, sz)], iv1)
            da = pltpu.async_copy(a0_hbm.at[iv0], ra.at[pl.ds(0, sz)], sema)
            db = pltpu.async_copy(a1_hbm.at[iv1], rb.at[pl.ds(0, sz)], semb)
            da.wait()
            db.wait()

            def add_row(r, carry):
                for kk in range(dl):
                    col = kk * 16
                    ra[r, pl.ds(col, 16)] = (ra[r, pl.ds(col, 16)]
                                             + rb[r, pl.ds(col, 16)])
                return carry

            lax.fori_loop(0, sz, add_row, 0)
            pltpu.sync_copy(ra.at[pl.ds(0, sz)], out_hbm.at[pl.ds(b, sz)])

        def step(j, carry):
            do_chunk(base0 + j * CH, i0, i1, CH)
            return carry

        lax.fori_loop(0, nfull, step, 0)
        if rem:
            do_chunk(base0 + nfull * CH, i0r, i1r, rem)

    return k(a0, a1, e_src, e_dst)


def _tc_prescale(d0, d1, x):
    n, d = x.shape
    R = 1000
    assert n % R == 0

    def body(d0r, d1r, xr, xs_o, dinv_o):
        deg = jnp.maximum(d0r[:, 0:1] + d1r[:, 0:1] + 1.0, 1.0)
        dv = lax.rsqrt(deg)
        dinv_o[...] = dv
        xs_o[...] = xr[...] * dv

    return pl.pallas_call(
        body,
        grid=(n // R,),
        in_specs=[pl.BlockSpec((R, d), lambda i: (i, 0)),
                  pl.BlockSpec((R, d), lambda i: (i, 0)),
                  pl.BlockSpec((R, d), lambda i: (i, 0))],
        out_specs=[pl.BlockSpec((R, d), lambda i: (i, 0)),
                   pl.BlockSpec((R, 1), lambda i: (i, 0))],
        out_shape=[jax.ShapeDtypeStruct((n, d), jnp.float32),
                   jax.ShapeDtypeStruct((n, 1), jnp.float32)],
    )(d0, d1, x)


def _tc_mid(a0, a1, xs, dinv, W1, b1, W2):
    n, d = xs.shape
    h = W1.shape[1]
    R = 1000
    assert n % R == 0

    def body(a0r, a1r, xsr, dvr, w1r, b1r, w2r, ys_o):
        dv = dvr[...]
        px = dv * (a0r[...] + a1r[...] + xsr[...])
        z1 = jnp.maximum(
            jnp.dot(px, w1r[...], preferred_element_type=jnp.float32)
            + b1r[...], 0.0)
        y1 = jnp.dot(z1, w2r[...], preferred_element_type=jnp.float32)
        ys_o[...] = y1 * dv

    return pl.pallas_call(
        body,
        grid=(n // R,),
        in_specs=[pl.BlockSpec((R, d), lambda i: (i, 0)),
                  pl.BlockSpec((R, d), lambda i: (i, 0)),
                  pl.BlockSpec((R, d), lambda i: (i, 0)),
                  pl.BlockSpec((R, 1), lambda i: (i, 0)),
                  pl.BlockSpec((d, h), lambda i: (0, 0)),
                  pl.BlockSpec((1, h), lambda i: (0, 0)),
                  pl.BlockSpec((h, d), lambda i: (0, 0))],
        out_specs=pl.BlockSpec((R, d), lambda i: (i, 0)),
        out_shape=jax.ShapeDtypeStruct((n, d), jnp.float32),
    )(a0, a1, xs, dinv, W1, b1, W2)


def _tc_decoder_pre(a0, a1, ys, dinv, b2, fc1_t, fc1_b_half, fc1_bias):
    n, d = ys.shape
    R = 1000
    assert n % R == 0

    def body(a0r, a1r, ysr, dvr, b2r, wtr, wbr, fbr, o0, o1):
        z2 = dvr[...] * (a0r[...] + a1r[...] + ysr[...]) + b2r[...]
        o0[...] = jnp.dot(z2, wtr[...],
                          preferred_element_type=jnp.float32) + fbr[...]
        o1[...] = jnp.dot(z2, wbr[...], preferred_element_type=jnp.float32)

    return pl.pallas_call(
        body,
        grid=(n // R,),
        in_specs=[pl.BlockSpec((R, d), lambda i: (i, 0)),
                  pl.BlockSpec((R, d), lambda i: (i, 0)),
                  pl.BlockSpec((R, d), lambda i: (i, 0)),
                  pl.BlockSpec((R, 1), lambda i: (i, 0)),
                  pl.BlockSpec((1, d), lambda i: (0, 0)),
                  pl.BlockSpec((d, d), lambda i: (0, 0)),
                  pl.BlockSpec((d, d), lambda i: (0, 0)),
                  pl.BlockSpec((1, d), lambda i: (0, 0))],
        out_specs=[pl.BlockSpec((R, d), lambda i: (i, 0)),
                   pl.BlockSpec((R, d), lambda i: (i, 0))],
        out_shape=[jax.ShapeDtypeStruct((n, d), jnp.float32),
                   jax.ShapeDtypeStruct((n, d), jnp.float32)],
    )(a0, a1, ys, dinv, b2, fc1_t, fc1_b_half, fc1_bias)


def _tc_decoder_mlp(g, f2, b2, f3, b3, f4, b4):
    E, d = g.shape
    h2, h3 = f2.shape[1], f3.shape[1]
    EB = 2000
    assert E % EB == 0

    def body(gr, f2r, b2r, f3r, b3r, f4r, b4r, o):
        t = jnp.maximum(gr[...], 0.0)
        t = jnp.maximum(
            jnp.dot(t, f2r[...], preferred_element_type=jnp.float32)
            + b2r[...], 0.0)
        t = jnp.maximum(
            jnp.dot(t, f3r[...], preferred_element_type=jnp.float32)
            + b3r[...], 0.0)
        o[...] = jnp.dot(t, f4r[...],
                         preferred_element_type=jnp.float32) + b4r[...]

    return pl.pallas_call(
        body,
        grid=(E // EB,),
        in_specs=[pl.BlockSpec((EB, d), lambda i: (i, 0)),
                  pl.BlockSpec((d, h2), lambda i: (0, 0)),
                  pl.BlockSpec((1, h2), lambda i: (0, 0)),
                  pl.BlockSpec((h2, h3), lambda i: (0, 0)),
                  pl.BlockSpec((1, h3), lambda i: (0, 0)),
                  pl.BlockSpec((h3, 1), lambda i: (0, 0)),
                  pl.BlockSpec((1, 1), lambda i: (0, 0))],
        out_specs=pl.BlockSpec((EB, 1), lambda i: (i, 0)),
        out_shape=jax.ShapeDtypeStruct((E, 1), jnp.float32),
    )(g, f2, b2, f3, b3, f4, b4)


def kernel(x, edge_index, W1, b1, W2, b2, fc1_W, fc1_b,
           fc2_W, fc2_b, fc3_W, fc3_b, fc4_W, fc4_b):
    n, d = x.shape
    ei = edge_index.astype(jnp.int32)
    e_src, e_dst = ei[0], ei[1]

    zerosd = jnp.zeros((n, d), jnp.float32)
    ones = jnp.ones((CH, d), jnp.float32)

    deg = _sc_degree(e_dst, ones, zerosd, n, d)
    xs, dinv = _tc_prescale(deg[0], deg[1], x)
    agg1 = _sc_aggregate(xs, e_src, e_dst, zerosd, n, d)
    ys = _tc_mid(agg1[0], agg1[1], xs, dinv, W1, b1.reshape(1, -1), W2)
    agg2 = _sc_aggregate(ys, e_src, e_dst, zerosd, n, d)
    a0, a1 = _tc_decoder_pre(agg2[0], agg2[1], ys, dinv, b2.reshape(1, -1),
                             fc1_W[:d], fc1_W[d:], fc1_b.reshape(1, -1))
    g = _sc_edge_gather(a0, a1, e_src, e_dst, d)
    out = _tc_decoder_mlp(g, fc2_W, fc2_b.reshape(1, -1),
                          fc3_W, fc3_b.reshape(1, -1),
                          fc4_W, fc4_b.reshape(1, -1))
    return jnp.squeeze(out, axis=-1)
